# baseline (speedup 1.0000x reference)
import jax
import jax.numpy as jnp
from jax.experimental import pallas as pl
from jax.experimental.pallas import tpu as pltpu

_NUM_TILES = 64
_LAMBDA = 0.5
_EPS = 1e-12
_STEPS = 8
_NBUF = 2


def _wm_kernel(q_hbm, sig_ref, pos_ref, out_ref, buf, sems):
    ch = buf.shape[1]

    sig = sig_ref[:]
    sig_inv = 1.0 / jnp.maximum(
        jnp.sqrt(jnp.sum(sig * sig, axis=1)), _EPS)
    tiles = jax.lax.broadcasted_iota(
        jnp.int32, (1, _NUM_TILES), 1).astype(jnp.float32)

    def issue(i, slot):
        pltpu.make_async_copy(
            q_hbm.at[pl.ds(i * ch, ch), :], buf.at[slot], sems.at[slot]
        ).start()

    for i in range(_NBUF):
        issue(i, i)

    for i in range(_STEPS):
        slot = i % _NBUF
        pltpu.make_async_copy(
            q_hbm.at[pl.ds(i * ch, ch), :], buf.at[slot], sems.at[slot]
        ).wait()
        q = buf[slot]
        dot = jax.lax.dot_general(
            q, sig, (((1,), (1,)), ((), ())),
            preferred_element_type=jnp.float32)
        q_inv = 1.0 / jnp.maximum(
            jnp.sqrt(jnp.sum(q * q, axis=1, keepdims=True)), _EPS)
        cos = dot * q_inv * sig_inv[None, :]
        pos = pos_ref[pl.ds(i * ch, ch), :]
        d_temporal = jnp.abs(pos - tiles) * (2.0 / (_NUM_TILES - 1))
        out_ref[pl.ds(i * ch, ch), :] = (
            (1.0 - _LAMBDA) * (1.0 - cos) + _LAMBDA * d_temporal)
        if i + _NBUF < _STEPS:
            issue(i + _NBUF, slot)


def kernel(query, signatures, query_pos):
    n, k = query.shape
    pos_f = query_pos.astype(jnp.float32).reshape(n, 1)
    return pl.pallas_call(
        _wm_kernel,
        in_specs=[
            pl.BlockSpec(memory_space=pltpu.HBM),
            pl.BlockSpec((_NUM_TILES, k), lambda: (0, 0)),
            pl.BlockSpec((n, 1), lambda: (0, 0)),
        ],
        out_specs=pl.BlockSpec((n, _NUM_TILES), lambda: (0, 0)),
        out_shape=jax.ShapeDtypeStruct((n, _NUM_TILES), jnp.float32),
        scratch_shapes=[
            pltpu.VMEM((_NBUF, n // _STEPS, k), jnp.float32),
            pltpu.SemaphoreType.DMA((_NBUF,)),
        ],
    )(query, signatures, pos_f)


# speedup vs baseline: 1.0562x; 1.0562x over previous
import jax
import jax.numpy as jnp
from jax.experimental import pallas as pl
from jax.experimental.pallas import tpu as pltpu

_NUM_TILES = 64
_LAMBDA = 0.5
_EPS = 1e-12
_BLOCK_M = 1024


def _wm_block_kernel(q_ref, sig_ref, pos_ref, out_ref):
    sig = sig_ref[:]
    sig_inv = 1.0 / jnp.maximum(
        jnp.sqrt(jnp.sum(sig * sig, axis=1)), _EPS)

    q = q_ref[:]
    dot = jax.lax.dot_general(
        q, sig, (((1,), (1,)), ((), ())),
        preferred_element_type=jnp.float32)
    ones = jnp.ones((8, q.shape[1]), jnp.float32)
    q2sum = jax.lax.dot_general(
        q * q, ones, (((1,), (1,)), ((), ())),
        preferred_element_type=jnp.float32)
    q_inv = 1.0 / jnp.maximum(jnp.sqrt(q2sum[:, :1]), _EPS)
    cos = dot * q_inv * sig_inv[None, :]

    pos = pos_ref[:]
    tiles = jax.lax.broadcasted_iota(
        jnp.int32, (1, _NUM_TILES), 1).astype(jnp.float32)
    d_temporal = jnp.abs(pos - tiles) * (2.0 / (_NUM_TILES - 1))

    out_ref[:] = (1.0 - _LAMBDA) * (1.0 - cos) + _LAMBDA * d_temporal


def kernel(query, signatures, query_pos):
    n, k = query.shape
    pos_f = query_pos.astype(jnp.float32).reshape(n, 1)
    grid = (n // _BLOCK_M,)
    return pl.pallas_call(
        _wm_block_kernel,
        grid=grid,
        in_specs=[
            pl.BlockSpec((_BLOCK_M, k), lambda i: (i, 0)),
            pl.BlockSpec((_NUM_TILES, k), lambda i: (0, 0)),
            pl.BlockSpec((_BLOCK_M, 1), lambda i: (i, 0)),
        ],
        out_specs=pl.BlockSpec((_BLOCK_M, _NUM_TILES), lambda i: (i, 0)),
        out_shape=jax.ShapeDtypeStruct((n, _NUM_TILES), jnp.float32),
        compiler_params=pltpu.CompilerParams(
            dimension_semantics=("parallel",)),
    )(query, signatures, pos_f)
